# Initial kernel scaffold; baseline (speedup 1.0000x reference)
import jax
import jax.numpy as jnp
from jax.experimental import pallas as pl


def kernel(Gu, Gi, F, edge_features, item_features, edge_index, user_idx, item_idx):
    raise NotImplementedError("write your pallas kernel here")


# speedup vs baseline: 1.6636x; 1.6636x over previous
import functools

import jax
import jax.numpy as jnp
from jax.experimental import pallas as pl
from jax.experimental.pallas import tpu as pltpu


def _dot_body(a_ref, b_ref, o_ref):
    o_ref[...] = jnp.sum(a_ref[...] * b_ref[...], axis=1, keepdims=True)


def _batched_dot(a, b):
    B, D = a.shape
    blk = 1024
    return pl.pallas_call(
        _dot_body,
        out_shape=jax.ShapeDtypeStruct((B, 1), jnp.float32),
        grid=(B // blk,),
        in_specs=[
            pl.BlockSpec((blk, D), lambda i: (i, 0)),
            pl.BlockSpec((blk, D), lambda i: (i, 0)),
        ],
        out_specs=pl.BlockSpec((blk, 1), lambda i: (i, 0)),
    )(a, b)


def kernel(Gu, Gi, F, edge_features, item_features, edge_index, user_idx, item_idx):
    NU = Gu.shape[0]
    NI = Gi.shape[0]
    E = edge_features.shape[0]
    u = edge_index[0, :E]
    items = edge_index[1, :E] - NU

    deg = jax.ops.segment_sum(jnp.ones((E,), jnp.float32), u, num_segments=NU)
    dinv = jnp.where(deg > 0, 1.0 / deg, 0.0)

    EE = edge_features @ F
    IF2 = item_features @ F
    s = dinv[u]
    Ci = jax.ops.segment_sum((0.7 * s)[:, None] * EE, items, num_segments=NI)
    Cu = jax.ops.segment_sum(0.8 * IF2[items], u, num_segments=NU)

    xu, xi = Gu, Gi
    au, ai = Gu, Gi
    for layer in range(3):
        yu = 0.3 * dinv[:, None] * xu
        yi = 0.2 * xi
        xi_new = Ci + jax.ops.segment_sum(yu[u], items, num_segments=NI)
        xu_new = Cu + jax.ops.segment_sum(yi[items], u, num_segments=NU)
        xu, xi = xu_new, xi_new
        au = au + xu * (1.0 / (layer + 2))
        ai = ai + xi * (1.0 / (layer + 2))

    ga = au[user_idx]
    gb = ai[item_idx]
    return _batched_dot(ga, gb)[:, 0]


# speedup vs baseline: 4.1941x; 2.5211x over previous
import functools

import jax
import jax.numpy as jnp
from jax import lax
from jax.experimental import pallas as pl
from jax.experimental.pallas import tpu as pltpu
from jax.experimental.pallas import tpu_sc as plsc

NU = 25000
NI = 25000
E = 400000
D = 64
HW = 32

NC = 2
NS = 16
NW = NC * NS
NPAD = 25088
STRIPE = NPAD // NS
EPAD = 401408
TCH = EPAD // NW
CHK = 128
NCHK = TCH // CHK
TRASH = NPAD - 1

_mesh = functools.partial(
    plsc.VectorSubcoreMesh, core_axis_name="c", subcore_axis_name="s",
    num_cores=NC, num_subcores=NS)

_params = pltpu.CompilerParams(use_tc_tiling_on_sc=False,
                               needs_layout_passes=False)


def _fill_rows(rows, width, value):
    nv = width // 16

    def zb(i, carry):
        rows[i // nv, pl.ds((i % nv) * 16, 16)] = jnp.full((16,), value, jnp.float32)
        return carry
    lax.fori_loop(0, CHK * nv, zb, 0)


def _zero_acc_stripe(zrows, acc, base):
    for t in range(STRIPE // 112):
        pltpu.sync_copy(zrows.at[pl.ds(0, 112)],
                        acc.at[pl.ds(base + t * 112, 112)])


def _deg_pass(dst1):
    W = 16

    @functools.partial(
        pl.kernel,
        out_type=jax.ShapeDtypeStruct((NC, NPAD, W), jnp.float32),
        mesh=_mesh(),
        compiler_params=_params,
        scratch_types=[
            pltpu.VMEM((NCHK, CHK), jnp.int32),
            pltpu.VMEM((CHK, W), jnp.float32),
            pltpu.VMEM((CHK, W), jnp.float32),
            pltpu.VMEM_SHARED((NPAD, W), jnp.float32),
        ],
    )
    def k(didx_ref, out_ref, idx_d, zrows, orows, acc):
        c = lax.axis_index("c")
        s = lax.axis_index("s")
        wid = c * NS + s
        _fill_rows(zrows, W, 0.0)
        _fill_rows(orows, W, 1.0)
        base = s * STRIPE
        _zero_acc_stripe(zrows, acc, base)
        plsc.subcore_barrier()

        def eb(kk, carry):
            pltpu.sync_copy(didx_ref.at[pl.ds(wid * TCH + kk * CHK, CHK)],
                            idx_d.at[kk])
            pltpu.sync_copy(orows, acc.at[idx_d.at[kk]], add=True)
            return carry
        lax.fori_loop(0, NCHK, eb, 0)
        plsc.subcore_barrier()
        pltpu.sync_copy(acc.at[pl.ds(base, STRIPE)],
                        out_ref.at[c, pl.ds(base, STRIPE)])
    return k(dst1)


def _gather_scalar(table, idx1):
    @functools.partial(
        pl.kernel,
        out_type=jax.ShapeDtypeStruct((EPAD,), jnp.float32),
        mesh=_mesh(),
        compiler_params=_params,
        scratch_types=[
            pltpu.VMEM((NPAD,), jnp.float32),
            pltpu.VMEM((TCH,), jnp.int32),
            pltpu.VMEM((TCH,), jnp.float32),
        ],
    )
    def k(table_ref, idx_ref, out_ref, tab_v, idx_v, s_v):
        c = lax.axis_index("c")
        s = lax.axis_index("s")
        wid = c * NS + s
        pltpu.sync_copy(table_ref, tab_v)
        pltpu.sync_copy(idx_ref.at[pl.ds(wid * TCH, TCH)], idx_v)

        def gb(j, carry):
            idx16 = idx_v[pl.ds(j * 16, 16)]
            s_v[pl.ds(j * 16, 16)] = plsc.load_gather(tab_v, [idx16])
            return carry
        lax.fori_loop(0, TCH // 16, gb, 0)
        pltpu.sync_copy(s_v, out_ref.at[pl.ds(wid * TCH, TCH)])
    return k(table, idx1)


def _edge_pass_gather(table, src1, dst1, tok):
    @functools.partial(
        pl.kernel,
        out_type=jax.ShapeDtypeStruct((NC, NPAD, HW), jnp.float32),
        mesh=_mesh(),
        compiler_params=_params,
        scratch_types=[
            pltpu.VMEM((NCHK, CHK), jnp.int32),
            pltpu.VMEM((NCHK, CHK), jnp.int32),
            pltpu.VMEM((CHK, HW), jnp.float32),
            pltpu.VMEM((16,), jnp.float32),
            pltpu.VMEM_SHARED((NPAD, HW), jnp.float32),
            pltpu.SemaphoreType.DMA,
        ],
    )
    def k(table_ref, sidx_ref, didx_ref, tok_ref, out_ref, idx_s, idx_d, rows,
          tok_v, acc, sem):
        c = lax.axis_index("c")
        s = lax.axis_index("s")
        wid = c * NS + s
        pltpu.sync_copy(tok_ref, tok_v)
        _fill_rows(rows, HW, 0.0)
        base = s * STRIPE
        _zero_acc_stripe(rows, acc, base)
        plsc.subcore_barrier()

        def eb(kk, carry):
            off = wid * TCH + kk * CHK
            pltpu.sync_copy(sidx_ref.at[pl.ds(off, CHK)], idx_s.at[kk])
            pltpu.sync_copy(didx_ref.at[pl.ds(off, CHK)], idx_d.at[kk])
            pltpu.async_copy(table_ref.at[idx_s.at[kk]], rows, sem).wait()
            pltpu.sync_copy(rows, acc.at[idx_d.at[kk]], add=True)
            return carry
        lax.fori_loop(0, NCHK, eb, 0)
        plsc.subcore_barrier()
        pltpu.sync_copy(acc.at[pl.ds(base, STRIPE)],
                        out_ref.at[c, pl.ds(base, STRIPE)])
    return k(table, src1, dst1, tok)


def _edge_pass_linear(table, dst1, tok):
    @functools.partial(
        pl.kernel,
        out_type=jax.ShapeDtypeStruct((NC, NPAD, HW), jnp.float32),
        mesh=_mesh(),
        compiler_params=_params,
        scratch_types=[
            pltpu.VMEM((NCHK, CHK), jnp.int32),
            pltpu.VMEM((CHK, HW), jnp.float32),
            pltpu.VMEM((16,), jnp.float32),
            pltpu.VMEM_SHARED((NPAD, HW), jnp.float32),
        ],
    )
    def k(table_ref, didx_ref, tok_ref, out_ref, idx_d, rows, tok_v, acc):
        c = lax.axis_index("c")
        s = lax.axis_index("s")
        wid = c * NS + s
        pltpu.sync_copy(tok_ref, tok_v)
        _fill_rows(rows, HW, 0.0)
        base = s * STRIPE
        _zero_acc_stripe(rows, acc, base)
        plsc.subcore_barrier()

        def eb(kk, carry):
            off = wid * TCH + kk * CHK
            pltpu.sync_copy(didx_ref.at[pl.ds(off, CHK)], idx_d.at[kk])
            pltpu.sync_copy(table_ref.at[pl.ds(off, CHK)], rows)
            pltpu.sync_copy(rows, acc.at[idx_d.at[kk]], add=True)
            return carry
        lax.fori_loop(0, NCHK, eb, 0)
        plsc.subcore_barrier()
        pltpu.sync_copy(acc.at[pl.ds(base, STRIPE)],
                        out_ref.at[c, pl.ds(base, STRIPE)])
    return k(table, dst1, tok)


def _edge_pass64(table, src1, dst1, tok):
    lo = _edge_pass_gather(table[:, :HW], src1, dst1, tok)
    hi = _edge_pass_gather(table[:, HW:], src1, dst1, lo[0, 0, :16])
    return jnp.concatenate([lo, hi], axis=2)


def _gather_rows(table_a, table_b, idx_a, idx_b, batch):
    per_w = batch // NW
    nchk = per_w // CHK

    @functools.partial(
        pl.kernel,
        out_type=(jax.ShapeDtypeStruct((batch, D), jnp.float32),
                  jax.ShapeDtypeStruct((batch, D), jnp.float32)),
        mesh=_mesh(),
        compiler_params=_params,
        scratch_types=[
            pltpu.VMEM((nchk, CHK), jnp.int32),
            pltpu.VMEM((nchk, CHK), jnp.int32),
            pltpu.VMEM((CHK, D), jnp.float32),
            pltpu.SemaphoreType.DMA,
        ],
    )
    def k(ta_ref, tb_ref, ia_ref, ib_ref, oa_ref, ob_ref, ia, ib, rows, sem):
        c = lax.axis_index("c")
        s = lax.axis_index("s")
        wid = c * NS + s

        def eb(kk, carry):
            off = wid * per_w + kk * CHK
            pltpu.sync_copy(ia_ref.at[pl.ds(off, CHK)], ia.at[kk])
            pltpu.sync_copy(ib_ref.at[pl.ds(off, CHK)], ib.at[kk])
            pltpu.async_copy(ta_ref.at[ia.at[kk]], rows, sem).wait()
            pltpu.sync_copy(rows, oa_ref.at[pl.ds(off, CHK)])
            pltpu.async_copy(tb_ref.at[ib.at[kk]], rows, sem).wait()
            pltpu.sync_copy(rows, ob_ref.at[pl.ds(off, CHK)])
            return carry
        lax.fori_loop(0, nchk, eb, 0)
    return k(table_a, table_b, idx_a, idx_b)


def _dot_body(a_ref, b_ref, o_ref):
    o_ref[...] = jnp.sum(a_ref[...] * b_ref[...], axis=1, keepdims=True)


def _batched_dot(a, b):
    B, Dd = a.shape
    blk = 1024
    return pl.pallas_call(
        _dot_body,
        out_shape=jax.ShapeDtypeStruct((B, 1), jnp.float32),
        grid=(B // blk,),
        in_specs=[
            pl.BlockSpec((blk, Dd), lambda i: (i, 0)),
            pl.BlockSpec((blk, Dd), lambda i: (i, 0)),
        ],
        out_specs=pl.BlockSpec((blk, 1), lambda i: (i, 0)),
    )(a, b)


def _pad_rows(x, n):
    return jnp.zeros((n, x.shape[1]), x.dtype).at[: x.shape[0]].set(x)


def kernel(Gu, Gi, F, edge_features, item_features, edge_index, user_idx, item_idx):
    u = edge_index[0, :E]
    items = edge_index[1, :E] - NU

    u_src = jnp.full((EPAD,), NU, jnp.int32).at[:E].set(u)
    it_src = jnp.full((EPAD,), NI, jnp.int32).at[:E].set(items)
    u_dst = jnp.full((EPAD,), TRASH, jnp.int32).at[:E].set(u)
    it_dst = jnp.full((EPAD,), TRASH, jnp.int32).at[:E].set(items)

    degp = _deg_pass(u_dst)
    deg = degp[0, :, 0] + degp[1, :, 0]
    dinv_pad = jnp.where(deg > 0, 1.0 / deg, 0.0)

    s = _gather_scalar(dinv_pad, u_src)[:E]
    EEs = (edge_features @ F) * (0.7 * s)[:, None]
    IF2s = (item_features @ F) * 0.8
    EEs_pad = _pad_rows(EEs, EPAD)
    IF2s_pad = _pad_rows(IF2s, NPAD)

    ci_lo = _edge_pass_linear(EEs_pad[:, :HW], it_dst, s[:16])
    ci_hi = _edge_pass_linear(EEs_pad[:, HW:], it_dst, ci_lo[0, 0, :16])
    ci = jnp.concatenate([ci_lo, ci_hi], axis=2)
    Ci = ci[0] + ci[1]
    cu = _edge_pass64(IF2s_pad, it_src, u_dst, ci_hi[0, 0, :16])
    Cu = cu[0] + cu[1]

    xu = _pad_rows(Gu, NPAD)
    xi = _pad_rows(Gi, NPAD)
    au, ai = xu, xi
    prev = cu
    for layer in range(3):
        yu = 0.3 * dinv_pad[:, None] * xu
        yi = 0.2 * xi
        pi = _edge_pass64(yu, u_src, it_dst, prev[0, 0, :16])
        pu = _edge_pass64(yi, it_src, u_dst, pi[0, 0, :16])
        prev = pu
        xi = Ci + pi[0] + pi[1]
        xu = Cu + pu[0] + pu[1]
        alpha = 1.0 / (layer + 2)
        au = au + xu * alpha
        ai = ai + xi * alpha

    ga, gb = _gather_rows(au, ai, user_idx.astype(jnp.int32),
                          item_idx.astype(jnp.int32), user_idx.shape[0])
    return _batched_dot(ga, gb)[:, 0]


# speedup vs baseline: 7.4344x; 1.7726x over previous
import functools

import jax
import jax.numpy as jnp
from jax import lax
from jax.experimental import pallas as pl
from jax.experimental.pallas import tpu as pltpu
from jax.experimental.pallas import tpu_sc as plsc

NU = 25000
NI = 25000
E = 400000
D = 64
HW = 32

NC = 2
NS = 16
NW = NC * NS
NPAD = 25088
STRIPE = NPAD // NS
EPAD = 401408
TCH = EPAD // NW
CHK = 128
NCHK = TCH // CHK
TRASH = NPAD - 1

_mesh = functools.partial(
    plsc.VectorSubcoreMesh, core_axis_name="c", subcore_axis_name="s",
    num_cores=NC, num_subcores=NS)

_params = pltpu.CompilerParams(use_tc_tiling_on_sc=False,
                               needs_layout_passes=False)


def _fill_rows(rows, width, value):
    nv = width // 16

    def zb(i, carry):
        rows[i // nv, pl.ds((i % nv) * 16, 16)] = jnp.full((16,), value, jnp.float32)
        return carry
    lax.fori_loop(0, CHK * nv, zb, 0)


def _zero_acc_stripe(zrows, acc, base):
    for t in range(STRIPE // 112):
        pltpu.sync_copy(zrows.at[pl.ds(0, 112)],
                        acc.at[pl.ds(base + t * 112, 112)])


def _deg_pass(dst2):
    W = 16

    @functools.partial(
        pl.kernel,
        out_type=jax.ShapeDtypeStruct((NC, NPAD, W), jnp.float32),
        mesh=_mesh(),
        compiler_params=_params,
        scratch_types=[
            pltpu.VMEM((NCHK, CHK), jnp.int32),
            pltpu.VMEM((CHK, W), jnp.float32),
            pltpu.VMEM((CHK, W), jnp.float32),
            pltpu.VMEM_SHARED((NPAD, W), jnp.float32),
        ],
    )
    def k(didx_ref, out_ref, idx_d, zrows, orows, acc):
        c = lax.axis_index("c")
        s = lax.axis_index("s")
        wid = c * NS + s
        _fill_rows(zrows, W, 0.0)
        _fill_rows(orows, W, 1.0)
        base = s * STRIPE
        _zero_acc_stripe(zrows, acc, base)
        plsc.subcore_barrier()
        pltpu.sync_copy(didx_ref.at[pl.ds(wid * NCHK, NCHK)], idx_d)

        def eb(kk, carry):
            pltpu.sync_copy(orows, acc.at[idx_d.at[kk]], add=True)
            return carry
        lax.fori_loop(0, NCHK, eb, 0)
        plsc.subcore_barrier()
        pltpu.sync_copy(acc.at[pl.ds(base, STRIPE)],
                        out_ref.at[c, pl.ds(base, STRIPE)])
    return k(dst2)


def _gather_scalar(table, idx2):
    @functools.partial(
        pl.kernel,
        out_type=jax.ShapeDtypeStruct((EPAD,), jnp.float32),
        mesh=_mesh(),
        compiler_params=_params,
        scratch_types=[
            pltpu.VMEM((NPAD,), jnp.float32),
            pltpu.VMEM((NCHK, CHK), jnp.int32),
            pltpu.VMEM((TCH,), jnp.float32),
        ],
    )
    def k(table_ref, idx_ref, out_ref, tab_v, idx_v, s_v):
        c = lax.axis_index("c")
        s = lax.axis_index("s")
        wid = c * NS + s
        pltpu.sync_copy(table_ref, tab_v)
        pltpu.sync_copy(idx_ref.at[pl.ds(wid * NCHK, NCHK)], idx_v)
        nv = CHK // 16

        def gb(j, carry):
            idx16 = idx_v[j // nv, pl.ds((j % nv) * 16, 16)]
            s_v[pl.ds(j * 16, 16)] = plsc.load_gather(tab_v, [idx16])
            return carry
        lax.fori_loop(0, TCH // 16, gb, 0)
        pltpu.sync_copy(s_v, out_ref.at[pl.ds(wid * TCH, TCH)])
    return k(table, idx2)


def _edge_pass_gather(table, src2, dst2, tok):
    @functools.partial(
        pl.kernel,
        out_type=jax.ShapeDtypeStruct((NC, NPAD, HW), jnp.float32),
        mesh=_mesh(),
        compiler_params=_params,
        scratch_types=[
            pltpu.VMEM((NCHK, CHK), jnp.int32),
            pltpu.VMEM((NCHK, CHK), jnp.int32),
            pltpu.VMEM((CHK, HW), jnp.float32),
            pltpu.VMEM((CHK, HW), jnp.float32),
            pltpu.VMEM((16,), jnp.float32),
            pltpu.VMEM_SHARED((NPAD, HW), jnp.float32),
            pltpu.SemaphoreType.DMA,
            pltpu.SemaphoreType.DMA,
        ],
    )
    def k(table_ref, sidx_ref, didx_ref, tok_ref, out_ref, idx_s, idx_d,
          rows0, rows1, tok_v, acc, sem0, sem1):
        c = lax.axis_index("c")
        s = lax.axis_index("s")
        wid = c * NS + s
        pltpu.sync_copy(tok_ref, tok_v)
        _fill_rows(rows0, HW, 0.0)
        base = s * STRIPE
        _zero_acc_stripe(rows0, acc, base)
        plsc.subcore_barrier()
        wb = wid * NCHK
        pltpu.sync_copy(sidx_ref.at[pl.ds(wb, NCHK)], idx_s)
        pltpu.sync_copy(didx_ref.at[pl.ds(wb, NCHK)], idx_d)

        bufs = (rows0, rows1)
        sems = (sem0, sem1)
        pltpu.async_copy(table_ref.at[idx_s.at[0]], rows0, sem0)

        def gb(g, carry):
            for j in range(2):
                kk = g * 2 + j
                cur, csem = bufs[j], sems[j]
                nxt, nsem = bufs[1 - j], sems[1 - j]

                @pl.when(kk + 1 < NCHK)
                def _():
                    pltpu.async_copy(table_ref.at[idx_s.at[kk + 1]], nxt, nsem)
                pltpu.make_async_copy(table_ref.at[idx_s.at[kk]], cur, csem).wait()
                pltpu.sync_copy(cur, acc.at[idx_d.at[kk]], add=True)
            return carry
        lax.fori_loop(0, NCHK // 2, gb, 0)
        plsc.subcore_barrier()
        pltpu.sync_copy(acc.at[pl.ds(base, STRIPE)],
                        out_ref.at[c, pl.ds(base, STRIPE)])
    return k(table, src2, dst2, tok)


def _edge_pass_linear(table, dst2, tok):
    @functools.partial(
        pl.kernel,
        out_type=jax.ShapeDtypeStruct((NC, NPAD, HW), jnp.float32),
        mesh=_mesh(),
        compiler_params=_params,
        scratch_types=[
            pltpu.VMEM((NCHK, CHK), jnp.int32),
            pltpu.VMEM((CHK, HW), jnp.float32),
            pltpu.VMEM((CHK, HW), jnp.float32),
            pltpu.VMEM((16,), jnp.float32),
            pltpu.VMEM_SHARED((NPAD, HW), jnp.float32),
            pltpu.SemaphoreType.DMA,
            pltpu.SemaphoreType.DMA,
        ],
    )
    def k(table_ref, didx_ref, tok_ref, out_ref, idx_d, rows0, rows1, tok_v,
          acc, sem0, sem1):
        c = lax.axis_index("c")
        s = lax.axis_index("s")
        wid = c * NS + s
        pltpu.sync_copy(tok_ref, tok_v)
        _fill_rows(rows0, HW, 0.0)
        base = s * STRIPE
        _zero_acc_stripe(rows0, acc, base)
        plsc.subcore_barrier()
        pltpu.sync_copy(didx_ref.at[pl.ds(wid * NCHK, NCHK)], idx_d)

        eb0 = wid * TCH
        bufs = (rows0, rows1)
        sems = (sem0, sem1)
        pltpu.async_copy(table_ref.at[pl.ds(eb0, CHK)], rows0, sem0)

        def gb(g, carry):
            for j in range(2):
                kk = g * 2 + j
                cur, csem = bufs[j], sems[j]
                nxt, nsem = bufs[1 - j], sems[1 - j]

                @pl.when(kk + 1 < NCHK)
                def _():
                    pltpu.async_copy(
                        table_ref.at[pl.ds(eb0 + (kk + 1) * CHK, CHK)], nxt, nsem)
                pltpu.make_async_copy(
                    table_ref.at[pl.ds(eb0 + kk * CHK, CHK)], cur, csem).wait()
                pltpu.sync_copy(cur, acc.at[idx_d.at[kk]], add=True)
            return carry
        lax.fori_loop(0, NCHK // 2, gb, 0)
        plsc.subcore_barrier()
        pltpu.sync_copy(acc.at[pl.ds(base, STRIPE)],
                        out_ref.at[c, pl.ds(base, STRIPE)])
    return k(table, dst2, tok)


def _edge_pass64(table, src2, dst2, tok):
    lo = _edge_pass_gather(table[:, :HW], src2, dst2, tok)
    hi = _edge_pass_gather(table[:, HW:], src2, dst2, lo[0, 0, :16])
    return jnp.concatenate([lo, hi], axis=2)


def _gather_rows(table_a, table_b, idx_a2, idx_b2, batch):
    per_w = batch // NW
    nchk = per_w // CHK

    @functools.partial(
        pl.kernel,
        out_type=(jax.ShapeDtypeStruct((batch, D), jnp.float32),
                  jax.ShapeDtypeStruct((batch, D), jnp.float32)),
        mesh=_mesh(),
        compiler_params=_params,
        scratch_types=[
            pltpu.VMEM((nchk, CHK), jnp.int32),
            pltpu.VMEM((nchk, CHK), jnp.int32),
            pltpu.VMEM((CHK, D), jnp.float32),
            pltpu.VMEM((CHK, D), jnp.float32),
            pltpu.SemaphoreType.DMA,
            pltpu.SemaphoreType.DMA,
        ],
    )
    def k(ta_ref, tb_ref, ia_ref, ib_ref, oa_ref, ob_ref, ia, ib,
          rows0, rows1, sem0, sem1):
        c = lax.axis_index("c")
        s = lax.axis_index("s")
        wid = c * NS + s
        pltpu.sync_copy(ia_ref.at[pl.ds(wid * nchk, nchk)], ia)
        pltpu.sync_copy(ib_ref.at[pl.ds(wid * nchk, nchk)], ib)
        pltpu.async_copy(ta_ref.at[ia.at[0]], rows0, sem0)
        pltpu.async_copy(tb_ref.at[ib.at[0]], rows1, sem1)

        def eb(kk, carry):
            off = wid * per_w + kk * CHK
            pltpu.make_async_copy(ta_ref.at[ia.at[kk]], rows0, sem0).wait()
            pltpu.sync_copy(rows0, oa_ref.at[pl.ds(off, CHK)])

            @pl.when(kk + 1 < nchk)
            def _():
                pltpu.async_copy(ta_ref.at[ia.at[kk + 1]], rows0, sem0)
            pltpu.make_async_copy(tb_ref.at[ib.at[kk]], rows1, sem1).wait()
            pltpu.sync_copy(rows1, ob_ref.at[pl.ds(off, CHK)])

            @pl.when(kk + 1 < nchk)
            def _():
                pltpu.async_copy(tb_ref.at[ib.at[kk + 1]], rows1, sem1)
            return carry
        lax.fori_loop(0, nchk, eb, 0)
    return k(table_a, table_b, idx_a2, idx_b2)


def _dot_body(a_ref, b_ref, o_ref):
    o_ref[...] = jnp.sum(a_ref[...] * b_ref[...], axis=1, keepdims=True)


def _batched_dot(a, b):
    B, Dd = a.shape
    blk = 1024
    return pl.pallas_call(
        _dot_body,
        out_shape=jax.ShapeDtypeStruct((B, 1), jnp.float32),
        grid=(B // blk,),
        in_specs=[
            pl.BlockSpec((blk, Dd), lambda i: (i, 0)),
            pl.BlockSpec((blk, Dd), lambda i: (i, 0)),
        ],
        out_specs=pl.BlockSpec((blk, 1), lambda i: (i, 0)),
    )(a, b)


def _pad_rows(x, n):
    return jnp.zeros((n, x.shape[1]), x.dtype).at[: x.shape[0]].set(x)


def kernel(Gu, Gi, F, edge_features, item_features, edge_index, user_idx, item_idx):
    u = edge_index[0, :E]
    items = edge_index[1, :E] - NU

    u_src = jnp.full((EPAD,), NU, jnp.int32).at[:E].set(u).reshape(-1, CHK)
    it_src = jnp.full((EPAD,), NI, jnp.int32).at[:E].set(items).reshape(-1, CHK)
    u_dst = jnp.full((EPAD,), TRASH, jnp.int32).at[:E].set(u).reshape(-1, CHK)
    it_dst = jnp.full((EPAD,), TRASH, jnp.int32).at[:E].set(items).reshape(-1, CHK)

    degp = _deg_pass(u_dst)
    deg = degp[0, :, 0] + degp[1, :, 0]
    dinv_pad = jnp.where(deg > 0, 1.0 / deg, 0.0)

    s = _gather_scalar(dinv_pad, u_src)[:E]
    EEs = (edge_features @ F) * (0.7 * s)[:, None]
    IF2s = (item_features @ F) * 0.8
    EEs_pad = _pad_rows(EEs, EPAD)
    IF2s_pad = _pad_rows(IF2s, NPAD)

    ci_lo = _edge_pass_linear(EEs_pad[:, :HW], it_dst, s[:16])
    ci_hi = _edge_pass_linear(EEs_pad[:, HW:], it_dst, ci_lo[0, 0, :16])
    ci = jnp.concatenate([ci_lo, ci_hi], axis=2)
    Ci = ci[0] + ci[1]
    cu = _edge_pass64(IF2s_pad, it_src, u_dst, ci_hi[0, 0, :16])
    Cu = cu[0] + cu[1]

    xu = _pad_rows(Gu, NPAD)
    xi = _pad_rows(Gi, NPAD)
    au, ai = xu, xi
    prev = cu
    for layer in range(3):
        yu = 0.3 * dinv_pad[:, None] * xu
        yi = 0.2 * xi
        pi = _edge_pass64(yu, u_src, it_dst, prev[0, 0, :16])
        pu = _edge_pass64(yi, it_src, u_dst, pi[0, 0, :16])
        prev = pu
        xi = Ci + pi[0] + pi[1]
        xu = Cu + pu[0] + pu[1]
        alpha = 1.0 / (layer + 2)
        au = au + xu * alpha
        ai = ai + xi * alpha

    ui2 = user_idx.astype(jnp.int32).reshape(-1, CHK)
    ii2 = item_idx.astype(jnp.int32).reshape(-1, CHK)
    ga, gb = _gather_rows(au, ai, ui2, ii2, user_idx.shape[0])
    return _batched_dot(ga, gb)[:, 0]
